# baseline (speedup 1.0000x reference)
import functools

import jax
import jax.numpy as jnp
from jax import lax
from jax.experimental import pallas as pl
from jax.experimental.pallas import tpu as pltpu, tpu_sc as plsc

N = 10000
E = 320000
D = 128
H = 128
Z = 128
B = 16384

NC = 2
NS = 16
NW = NC * NS

EK = 128
ENCH = 80
EPWP = ENCH * EK
EPAD = NW * EPWP - E
NP = N + 8

RPS = 624
ZR = 16

BPW = B // NW
BK = 128
BNCH = BPW // BK

NBUF = 2
CPP = 16
PASSES = ENCH // CPP

_mesh = plsc.VectorSubcoreMesh(core_axis_name="c", subcore_axis_name="s")


def _zero_vec():
    return jnp.zeros((16,), jnp.float32)


@functools.partial(
    pl.kernel,
    out_type=jax.ShapeDtypeStruct((NC * N,), jnp.float32),
    mesh=_mesh,
    scratch_types=[
        pltpu.VMEM_SHARED((NP,), jnp.float32),
        pltpu.VMEM((ENCH, EK), jnp.int32),
        pltpu.VMEM((EK,), jnp.float32),
        pltpu.VMEM((1024,), jnp.float32),
        pltpu.VMEM((1000,), jnp.float32),
        pltpu.SemaphoreType.DMA((4,)),
    ],
)
def _sc_degree(dstr_hbm, deg_hbm, acc_sh, dst_l, ones_v, zb_v, dg_v, ssem):
    NB = 4
    c = lax.axis_index("c")
    s = lax.axis_index("s")
    w = c * NS + s
    for k in range(64):
        zb_v[pl.ds(k * 16, 16)] = _zero_vec()
    for k in range(EK // 16):
        ones_v[pl.ds(k * 16, 16)] = jnp.ones((16,), jnp.float32)
    @pl.when(s < 10)
    def _():
        pltpu.sync_copy(zb_v.at[pl.ds(0, 1000)], acc_sh.at[pl.ds(s * 1000, 1000)])
    plsc.subcore_barrier()
    pltpu.sync_copy(dstr_hbm.at[w], dst_l)

    def body(j, carry):
        b = lax.rem(j, NB)

        @pl.when(j >= NB)
        def _():
            pltpu.make_async_copy(ones_v, acc_sh.at[dst_l.at[j - NB]],
                                  ssem.at[b]).wait()
        pltpu.async_copy(ones_v, acc_sh.at[dst_l.at[j]], ssem.at[b], add=True)
        return carry

    lax.fori_loop(0, ENCH, body, 0)
    for k in range(NB):
        j = ENCH - NB + k
        pltpu.make_async_copy(ones_v, acc_sh.at[dst_l.at[j]],
                              ssem.at[j % NB]).wait()
    plsc.subcore_barrier()

    @pl.when(s < 10)
    def _():
        pltpu.sync_copy(acc_sh.at[pl.ds(s * 1000, 1000)], dg_v)
        pltpu.sync_copy(dg_v, deg_hbm.at[pl.ds(c * N + s * 1000, 1000)])


@functools.partial(
    pl.kernel,
    out_type=jax.ShapeDtypeStruct((NC, N, D), jnp.float32),
    mesh=_mesh,
    scratch_types=[
        pltpu.VMEM_SHARED((NP, D), jnp.float32),
        pltpu.VMEM((CPP, EK), jnp.int32),
        pltpu.VMEM((CPP, EK), jnp.int32),
        pltpu.VMEM((NBUF, EK, D), jnp.float32),
        pltpu.VMEM((ZR, D), jnp.float32),
        pltpu.SemaphoreType.DMA((NBUF,)),
        pltpu.SemaphoreType.DMA((NBUF,)),
    ],
)
def _sc_scatter(hs_hbm, srcr_hbm, dstr_hbm, acc_hbm,
                acc_sh, src_l, dst_l, rows_v, zb_v, gsem, ssem):
    c = lax.axis_index("c")
    s = lax.axis_index("s")
    w = c * NS + s
    for i in range(ZR):
        for j in range(D // 16):
            zb_v[i, pl.ds(j * 16, 16)] = _zero_vec()

    base = s * RPS
    nch = jnp.where(s == NS - 1, 40, 39)

    def zbody(t, carry):
        pltpu.sync_copy(zb_v, acc_sh.at[pl.ds(base + t * ZR, ZR)])
        return carry
    lax.fori_loop(0, nch, zbody, 0)
    plsc.subcore_barrier()

    def pass_body(p, pcarry):
        pltpu.sync_copy(srcr_hbm.at[w, pl.ds(p * CPP, CPP)], src_l)
        pltpu.sync_copy(dstr_hbm.at[w, pl.ds(p * CPP, CPP)], dst_l)

        for b in range(NBUF - 1):
            pltpu.async_copy(hs_hbm.at[src_l.at[b]], rows_v.at[b], gsem.at[b])

        def body(j, carry):
            b = lax.rem(j, NBUF)
            c2 = j + NBUF - 1
            b2 = lax.rem(c2, NBUF)

            @pl.when(c2 < CPP)
            def _():
                @pl.when(j >= 1)
                def _():
                    pltpu.make_async_copy(rows_v.at[b2],
                                          acc_sh.at[dst_l.at[j - 1]],
                                          ssem.at[b2]).wait()
                pltpu.async_copy(hs_hbm.at[src_l.at[c2]], rows_v.at[b2],
                                 gsem.at[b2])
            pltpu.make_async_copy(hs_hbm.at[src_l.at[j]], rows_v.at[b],
                                  gsem.at[b]).wait()
            pltpu.async_copy(rows_v.at[b], acc_sh.at[dst_l.at[j]], ssem.at[b],
                             add=True)
            return carry

        lax.fori_loop(0, CPP, body, 0)
        for k in range(NBUF):
            j = CPP - NBUF + k
            pltpu.make_async_copy(rows_v.at[j % NBUF], acc_sh.at[dst_l.at[j]],
                                  ssem.at[j % NBUF]).wait()
        return pcarry

    lax.fori_loop(0, PASSES, pass_body, 0)
    plsc.subcore_barrier()

    def wbody(t, carry):
        pltpu.sync_copy(acc_sh.at[pl.ds(base + t * ZR, ZR)], zb_v)
        pltpu.sync_copy(zb_v, acc_hbm.at[c, pl.ds(base + t * ZR, ZR)])
        return carry
    lax.fori_loop(0, nch, wbody, 0)


_GNB = 4


@functools.partial(
    pl.kernel,
    out_type=[jax.ShapeDtypeStruct((B, Z), jnp.float32),
              jax.ShapeDtypeStruct((B, Z), jnp.float32)],
    mesh=_mesh,
    scratch_types=[
        pltpu.VMEM((2 * BNCH, BK), jnp.int32),
        pltpu.VMEM((_GNB, BK, Z), jnp.float32),
        pltpu.SemaphoreType.DMA((_GNB,)),
        pltpu.SemaphoreType.DMA((_GNB,)),
    ],
)
def _sc_gather(z_hbm, eir_hbm, ejr_hbm, zi_hbm, zj_hbm,
               idx_l, rows_v, gsem, wsem):
    c = lax.axis_index("c")
    s = lax.axis_index("s")
    w = c * NS + s
    pltpu.sync_copy(eir_hbm.at[w], idx_l.at[pl.ds(0, BNCH)])
    pltpu.sync_copy(ejr_hbm.at[w], idx_l.at[pl.ds(BNCH, BNCH)])

    NQ = 2 * BNCH

    def _out_slice(q):
        ref = zi_hbm if q < BNCH else zj_hbm
        return ref.at[pl.ds(w * BPW + (q % BNCH) * BK, BK)]

    for b in range(_GNB - 1):
        pltpu.async_copy(z_hbm.at[idx_l.at[b]], rows_v.at[b], gsem.at[b])
    for q in range(NQ):
        b = q % _GNB
        g = q + _GNB - 1
        if g < NQ:
            bg = g % _GNB
            if q >= 1:
                pltpu.make_async_copy(rows_v.at[bg], _out_slice(q - 1),
                                      wsem.at[bg]).wait()
            pltpu.async_copy(z_hbm.at[idx_l.at[g]], rows_v.at[bg],
                             gsem.at[bg])
        pltpu.make_async_copy(z_hbm.at[idx_l.at[q]], rows_v.at[b],
                              gsem.at[b]).wait()
        pltpu.async_copy(rows_v.at[b], _out_slice(q), wsem.at[b])
    for q in range(NQ - _GNB, NQ):
        pltpu.make_async_copy(rows_v.at[q % _GNB], _out_slice(q),
                              wsem.at[q % _GNB]).wait()


def _elu(x):
    return jnp.where(x > 0.0, x, jnp.exp(jnp.minimum(x, 0.0)) - 1.0)


def _tc_matmul_body(feat_ref, w1_ref, h_ref):
    h_ref[...] = jnp.dot(feat_ref[...], w1_ref[...],
                         preferred_element_type=jnp.float32)


def _tc_matmul(features, W1):
    return pl.pallas_call(
        _tc_matmul_body,
        out_shape=jax.ShapeDtypeStruct((N, D), jnp.float32),
    )(features, W1)


def _tc_prescale_body(h_ref, degt_ref, dinv_ref, hs1_ref):
    deg = degt_ref[:, 0:1] + degt_ref[:, 1:2] + 1.0
    dinv = lax.rsqrt(deg)
    dinv_ref[...] = dinv
    hs1_ref[0:N, :] = h_ref[...] * dinv
    hs1_ref[N:NP, :] = jnp.zeros((NP - N, D), jnp.float32)


def _tc_prescale(h, degt):
    return pl.pallas_call(
        _tc_prescale_body,
        out_shape=[jax.ShapeDtypeStruct((N, 1), jnp.float32),
                   jax.ShapeDtypeStruct((NP, D), jnp.float32)],
    )(h, degt)


def _combine(acc_ref, hs_ref, dinv_ref, b_ref, g_ref, be_ref):
    hs = hs_ref[...][0:N, :]
    pre = (acc_ref[0] + acc_ref[1] + hs) * dinv_ref[...] + b_ref[...]
    mean = jnp.mean(pre, axis=0, keepdims=True)
    var = jnp.mean((pre - mean) ** 2, axis=0, keepdims=True)
    o = (pre - mean) * lax.rsqrt(var + 1e-5) * g_ref[...] + be_ref[...]
    return _elu(o)


def _tc_mid_body(acc_ref, hs_ref, dinv_ref, b_ref, g_ref, be_ref, wn_ref,
                 o_ref, hsn_ref):
    o = _combine(acc_ref, hs_ref, dinv_ref, b_ref, g_ref, be_ref)
    o_ref[...] = o
    h = jnp.dot(o, wn_ref[...], preferred_element_type=jnp.float32)
    hsn_ref[0:N, :] = h * dinv_ref[...]
    hsn_ref[N:NP, :] = jnp.zeros((NP - N, D), jnp.float32)


def _tc_mid(acc, hs, dinv, b, g, be, Wn):
    return pl.pallas_call(
        _tc_mid_body,
        out_shape=[jax.ShapeDtypeStruct((N, D), jnp.float32),
                   jax.ShapeDtypeStruct((NP, D), jnp.float32)],
    )(acc, hs, dinv, b, g, be, Wn)


def _tc_final_body(acc_ref, hs_ref, dinv_ref, b_ref, g_ref, be_ref,
                   o1_ref, o2_ref, jk_ref, z_ref):
    o3 = _combine(acc_ref, hs_ref, dinv_ref, b_ref, g_ref, be_ref)
    jk = jk_ref[...]
    m = jnp.max(jk)
    e = jnp.exp(jk - m)
    wts = e / jnp.sum(e)
    z_ref[...] = (wts[0, 0] * o1_ref[...] + wts[0, 1] * o2_ref[...]
                  + wts[0, 2] * o3)


def _tc_final(acc, hs, dinv, b, g, be, o1, o2, jk_w):
    return pl.pallas_call(
        _tc_final_body,
        out_shape=jax.ShapeDtypeStruct((N, D), jnp.float32),
    )(acc, hs, dinv, b, g, be, o1, o2, jk_w)


_DB = 4096


def _tc_decoder_body(zi_ref, zj_ref, tf_ref, tcf_ref, wd1_ref, wlast_ref,
                     bd1_ref, wd2_ref, lf_ref, lcf_ref):
    zz = zi_ref[...] * zj_ref[...]
    gmat = jnp.dot(zz, wd1_ref[...], preferred_element_type=jnp.float32)
    gmat = gmat + bd1_ref[...]
    hf = _elu(gmat + tf_ref[...] * wlast_ref[...])
    hcf = _elu(gmat + tcf_ref[...] * wlast_ref[...])
    lf_ref[...] = jnp.dot(hf, wd2_ref[...], preferred_element_type=jnp.float32)
    lcf_ref[...] = jnp.dot(hcf, wd2_ref[...], preferred_element_type=jnp.float32)


def _tc_decoder(zi, zj, tf, tcf, Wd1a, wlast, bd1, Wd2):
    nblk = B // _DB
    return pl.pallas_call(
        _tc_decoder_body,
        grid=(nblk,),
        in_specs=[
            pl.BlockSpec((_DB, Z), lambda i: (i, 0)),
            pl.BlockSpec((_DB, Z), lambda i: (i, 0)),
            pl.BlockSpec((_DB, 1), lambda i: (i, 0)),
            pl.BlockSpec((_DB, 1), lambda i: (i, 0)),
            pl.BlockSpec((Z, H), lambda i: (0, 0)),
            pl.BlockSpec((1, H), lambda i: (0, 0)),
            pl.BlockSpec((1, H), lambda i: (0, 0)),
            pl.BlockSpec((H, 1), lambda i: (0, 0)),
        ],
        out_specs=[
            pl.BlockSpec((_DB, 1), lambda i: (i, 0)),
            pl.BlockSpec((_DB, 1), lambda i: (i, 0)),
        ],
        out_shape=[jax.ShapeDtypeStruct((B, 1), jnp.float32),
                   jax.ShapeDtypeStruct((B, 1), jnp.float32)],
    )(zi, zj, tf, tcf, Wd1a, wlast, bd1, Wd2)


def kernel(edge_index, features, edges, T_f_batch, T_cf_batch,
           W1, b1, W2, b2, W3, b3, g1, be1, g2, be2, g3, be3,
           jk_w, Wd1, bd1, Wd2):
    pad = jnp.full((EPAD,), N, jnp.int32)
    srcr = jnp.concatenate([edge_index[0], pad]).reshape(NW, ENCH, EK)
    dstr = jnp.concatenate([edge_index[1], pad]).reshape(NW, ENCH, EK)

    h1 = _tc_matmul(features, W1)
    degp = _sc_degree(dstr).reshape(NC, N)
    degt = degp.T
    dinv, hs = _tc_prescale(h1, degt)

    b1r, g1r, be1r = b1.reshape(1, H), g1.reshape(1, H), be1.reshape(1, H)
    b2r, g2r, be2r = b2.reshape(1, H), g2.reshape(1, H), be2.reshape(1, H)
    b3r, g3r, be3r = b3.reshape(1, Z), g3.reshape(1, Z), be3.reshape(1, Z)

    acc = _sc_scatter(hs, srcr, dstr)
    o1, hs = _tc_mid(acc, hs, dinv, b1r, g1r, be1r, W2)
    acc = _sc_scatter(hs, srcr, dstr)
    o2, hs = _tc_mid(acc, hs, dinv, b2r, g2r, be2r, W3)
    acc = _sc_scatter(hs, srcr, dstr)
    z = _tc_final(acc, hs, dinv, b3r, g3r, be3r, o1, o2, jk_w.reshape(1, 3))

    eir = edges[:, 0].reshape(NW, BNCH, BK)
    ejr = edges[:, 1].reshape(NW, BNCH, BK)
    zi, zj = _sc_gather(z, eir, ejr)
    logits_f, logits_cf = _tc_decoder(
        zi, zj, T_f_batch.reshape(B, 1), T_cf_batch.reshape(B, 1),
        Wd1[:Z], Wd1[Z:Z + 1], bd1.reshape(1, H), Wd2)
    return (z, logits_f.reshape(B), logits_cf.reshape(B))


# speedup vs baseline: 1.0261x; 1.0261x over previous
import functools

import jax
import jax.numpy as jnp
from jax import lax
from jax.experimental import pallas as pl
from jax.experimental.pallas import tpu as pltpu, tpu_sc as plsc

N = 10000
E = 320000
D = 128
H = 128
Z = 128
B = 16384

NC = 2
NS = 16
NW = NC * NS

EK = 80
ENCH = 128
EPWP = ENCH * EK
EPAD = NW * EPWP - E
NP = N + 8

RPS = 624
ZR = 16

BPW = B // NW
BK = 128
BNCH = BPW // BK

NBUF = 3
CPP = 32
PASSES = ENCH // CPP

_mesh = plsc.VectorSubcoreMesh(core_axis_name="c", subcore_axis_name="s")


def _zero_vec():
    return jnp.zeros((16,), jnp.float32)


@functools.partial(
    pl.kernel,
    out_type=jax.ShapeDtypeStruct((NC * N,), jnp.float32),
    mesh=_mesh,
    scratch_types=[
        pltpu.VMEM_SHARED((NP,), jnp.float32),
        pltpu.VMEM((ENCH, EK), jnp.int32),
        pltpu.VMEM((EK,), jnp.float32),
        pltpu.VMEM((1024,), jnp.float32),
        pltpu.VMEM((1000,), jnp.float32),
        pltpu.SemaphoreType.DMA((4,)),
    ],
)
def _sc_degree(dstr_hbm, deg_hbm, acc_sh, dst_l, ones_v, zb_v, dg_v, ssem):
    NB = 4
    c = lax.axis_index("c")
    s = lax.axis_index("s")
    w = c * NS + s
    for k in range(64):
        zb_v[pl.ds(k * 16, 16)] = _zero_vec()
    for k in range(EK // 16):
        ones_v[pl.ds(k * 16, 16)] = jnp.ones((16,), jnp.float32)
    @pl.when(s < 10)
    def _():
        pltpu.sync_copy(zb_v.at[pl.ds(0, 1000)], acc_sh.at[pl.ds(s * 1000, 1000)])
    plsc.subcore_barrier()
    pltpu.sync_copy(dstr_hbm.at[w], dst_l)

    def body(j, carry):
        b = lax.rem(j, NB)

        @pl.when(j >= NB)
        def _():
            pltpu.make_async_copy(ones_v, acc_sh.at[dst_l.at[j - NB]],
                                  ssem.at[b]).wait()
        pltpu.async_copy(ones_v, acc_sh.at[dst_l.at[j]], ssem.at[b], add=True)
        return carry

    lax.fori_loop(0, ENCH, body, 0)
    for k in range(NB):
        j = ENCH - NB + k
        pltpu.make_async_copy(ones_v, acc_sh.at[dst_l.at[j]],
                              ssem.at[j % NB]).wait()
    plsc.subcore_barrier()

    @pl.when(s < 10)
    def _():
        pltpu.sync_copy(acc_sh.at[pl.ds(s * 1000, 1000)], dg_v)
        pltpu.sync_copy(dg_v, deg_hbm.at[pl.ds(c * N + s * 1000, 1000)])


@functools.partial(
    pl.kernel,
    out_type=jax.ShapeDtypeStruct((NC, N, D), jnp.float32),
    mesh=_mesh,
    scratch_types=[
        pltpu.VMEM_SHARED((NP, D), jnp.float32),
        pltpu.VMEM((CPP, EK), jnp.int32),
        pltpu.VMEM((CPP, EK), jnp.int32),
        pltpu.VMEM((NBUF, EK, D), jnp.float32),
        pltpu.VMEM((ZR, D), jnp.float32),
        pltpu.SemaphoreType.DMA((NBUF,)),
        pltpu.SemaphoreType.DMA((NBUF,)),
    ],
)
def _sc_scatter(hs_hbm, srcr_hbm, dstr_hbm, acc_hbm,
                acc_sh, src_l, dst_l, rows_v, zb_v, gsem, ssem):
    c = lax.axis_index("c")
    s = lax.axis_index("s")
    w = c * NS + s
    for i in range(ZR):
        for j in range(D // 16):
            zb_v[i, pl.ds(j * 16, 16)] = _zero_vec()

    base = s * RPS
    nch = jnp.where(s == NS - 1, 40, 39)

    def zbody(t, carry):
        pltpu.sync_copy(zb_v, acc_sh.at[pl.ds(base + t * ZR, ZR)])
        return carry
    lax.fori_loop(0, nch, zbody, 0)
    plsc.subcore_barrier()

    def pass_body(p, pcarry):
        pltpu.sync_copy(srcr_hbm.at[w, pl.ds(p * CPP, CPP)], src_l)
        pltpu.sync_copy(dstr_hbm.at[w, pl.ds(p * CPP, CPP)], dst_l)

        for b in range(NBUF - 1):
            pltpu.async_copy(hs_hbm.at[src_l.at[b]], rows_v.at[b], gsem.at[b])

        def body(j, carry):
            b = lax.rem(j, NBUF)
            c2 = j + NBUF - 1
            b2 = lax.rem(c2, NBUF)

            @pl.when(c2 < CPP)
            def _():
                @pl.when(j >= 1)
                def _():
                    pltpu.make_async_copy(rows_v.at[b2],
                                          acc_sh.at[dst_l.at[j - 1]],
                                          ssem.at[b2]).wait()
                pltpu.async_copy(hs_hbm.at[src_l.at[c2]], rows_v.at[b2],
                                 gsem.at[b2])
            pltpu.make_async_copy(hs_hbm.at[src_l.at[j]], rows_v.at[b],
                                  gsem.at[b]).wait()
            pltpu.async_copy(rows_v.at[b], acc_sh.at[dst_l.at[j]], ssem.at[b],
                             add=True)
            return carry

        lax.fori_loop(0, CPP, body, 0)
        for k in range(NBUF):
            j = CPP - NBUF + k
            pltpu.make_async_copy(rows_v.at[j % NBUF], acc_sh.at[dst_l.at[j]],
                                  ssem.at[j % NBUF]).wait()
        return pcarry

    lax.fori_loop(0, PASSES, pass_body, 0)
    plsc.subcore_barrier()

    def wbody(t, carry):
        pltpu.sync_copy(acc_sh.at[pl.ds(base + t * ZR, ZR)], zb_v)
        pltpu.sync_copy(zb_v, acc_hbm.at[c, pl.ds(base + t * ZR, ZR)])
        return carry
    lax.fori_loop(0, nch, wbody, 0)


_GNB = 4


@functools.partial(
    pl.kernel,
    out_type=[jax.ShapeDtypeStruct((B, Z), jnp.float32),
              jax.ShapeDtypeStruct((B, Z), jnp.float32)],
    mesh=_mesh,
    scratch_types=[
        pltpu.VMEM((2 * BNCH, BK), jnp.int32),
        pltpu.VMEM((_GNB, BK, Z), jnp.float32),
        pltpu.SemaphoreType.DMA((_GNB,)),
        pltpu.SemaphoreType.DMA((_GNB,)),
    ],
)
def _sc_gather(z_hbm, eir_hbm, ejr_hbm, zi_hbm, zj_hbm,
               idx_l, rows_v, gsem, wsem):
    c = lax.axis_index("c")
    s = lax.axis_index("s")
    w = c * NS + s
    pltpu.sync_copy(eir_hbm.at[w], idx_l.at[pl.ds(0, BNCH)])
    pltpu.sync_copy(ejr_hbm.at[w], idx_l.at[pl.ds(BNCH, BNCH)])

    NQ = 2 * BNCH

    def _out_slice(q):
        ref = zi_hbm if q < BNCH else zj_hbm
        return ref.at[pl.ds(w * BPW + (q % BNCH) * BK, BK)]

    for b in range(_GNB - 1):
        pltpu.async_copy(z_hbm.at[idx_l.at[b]], rows_v.at[b], gsem.at[b])
    for q in range(NQ):
        b = q % _GNB
        g = q + _GNB - 1
        if g < NQ:
            bg = g % _GNB
            if q >= 1:
                pltpu.make_async_copy(rows_v.at[bg], _out_slice(q - 1),
                                      wsem.at[bg]).wait()
            pltpu.async_copy(z_hbm.at[idx_l.at[g]], rows_v.at[bg],
                             gsem.at[bg])
        pltpu.make_async_copy(z_hbm.at[idx_l.at[q]], rows_v.at[b],
                              gsem.at[b]).wait()
        pltpu.async_copy(rows_v.at[b], _out_slice(q), wsem.at[b])
    for q in range(NQ - _GNB, NQ):
        pltpu.make_async_copy(rows_v.at[q % _GNB], _out_slice(q),
                              wsem.at[q % _GNB]).wait()


def _elu(x):
    return jnp.where(x > 0.0, x, jnp.exp(jnp.minimum(x, 0.0)) - 1.0)


def _tc_matmul_body(feat_ref, w1_ref, h_ref):
    h_ref[...] = jnp.dot(feat_ref[...], w1_ref[...],
                         preferred_element_type=jnp.float32)


def _tc_matmul(features, W1):
    return pl.pallas_call(
        _tc_matmul_body,
        out_shape=jax.ShapeDtypeStruct((N, D), jnp.float32),
    )(features, W1)


def _tc_prescale_body(h_ref, degt_ref, dinv_ref, hs1_ref):
    deg = degt_ref[:, 0:1] + degt_ref[:, 1:2] + 1.0
    dinv = lax.rsqrt(deg)
    dinv_ref[...] = dinv
    hs1_ref[0:N, :] = h_ref[...] * dinv
    hs1_ref[N:NP, :] = jnp.zeros((NP - N, D), jnp.float32)


def _tc_prescale(h, degt):
    return pl.pallas_call(
        _tc_prescale_body,
        out_shape=[jax.ShapeDtypeStruct((N, 1), jnp.float32),
                   jax.ShapeDtypeStruct((NP, D), jnp.float32)],
    )(h, degt)


def _combine(acc_ref, hs_ref, dinv_ref, b_ref, g_ref, be_ref):
    hs = hs_ref[...][0:N, :]
    pre = (acc_ref[0] + acc_ref[1] + hs) * dinv_ref[...] + b_ref[...]
    mean = jnp.mean(pre, axis=0, keepdims=True)
    var = jnp.mean((pre - mean) ** 2, axis=0, keepdims=True)
    o = (pre - mean) * lax.rsqrt(var + 1e-5) * g_ref[...] + be_ref[...]
    return _elu(o)


def _tc_mid_body(acc_ref, hs_ref, dinv_ref, b_ref, g_ref, be_ref, wn_ref,
                 o_ref, hsn_ref):
    o = _combine(acc_ref, hs_ref, dinv_ref, b_ref, g_ref, be_ref)
    o_ref[...] = o
    h = jnp.dot(o, wn_ref[...], preferred_element_type=jnp.float32)
    hsn_ref[0:N, :] = h * dinv_ref[...]
    hsn_ref[N:NP, :] = jnp.zeros((NP - N, D), jnp.float32)


def _tc_mid(acc, hs, dinv, b, g, be, Wn):
    return pl.pallas_call(
        _tc_mid_body,
        out_shape=[jax.ShapeDtypeStruct((N, D), jnp.float32),
                   jax.ShapeDtypeStruct((NP, D), jnp.float32)],
    )(acc, hs, dinv, b, g, be, Wn)


def _tc_final_body(acc_ref, hs_ref, dinv_ref, b_ref, g_ref, be_ref,
                   o1_ref, o2_ref, jk_ref, z_ref):
    o3 = _combine(acc_ref, hs_ref, dinv_ref, b_ref, g_ref, be_ref)
    jk = jk_ref[...]
    m = jnp.max(jk)
    e = jnp.exp(jk - m)
    wts = e / jnp.sum(e)
    z_ref[...] = (wts[0, 0] * o1_ref[...] + wts[0, 1] * o2_ref[...]
                  + wts[0, 2] * o3)


def _tc_final(acc, hs, dinv, b, g, be, o1, o2, jk_w):
    return pl.pallas_call(
        _tc_final_body,
        out_shape=jax.ShapeDtypeStruct((N, D), jnp.float32),
    )(acc, hs, dinv, b, g, be, o1, o2, jk_w)


_DB = 4096


def _tc_decoder_body(zi_ref, zj_ref, tf_ref, tcf_ref, wd1_ref, wlast_ref,
                     bd1_ref, wd2_ref, lf_ref, lcf_ref):
    zz = zi_ref[...] * zj_ref[...]
    gmat = jnp.dot(zz, wd1_ref[...], preferred_element_type=jnp.float32)
    gmat = gmat + bd1_ref[...]
    hf = _elu(gmat + tf_ref[...] * wlast_ref[...])
    hcf = _elu(gmat + tcf_ref[...] * wlast_ref[...])
    lf_ref[...] = jnp.dot(hf, wd2_ref[...], preferred_element_type=jnp.float32)
    lcf_ref[...] = jnp.dot(hcf, wd2_ref[...], preferred_element_type=jnp.float32)


def _tc_decoder(zi, zj, tf, tcf, Wd1a, wlast, bd1, Wd2):
    nblk = B // _DB
    return pl.pallas_call(
        _tc_decoder_body,
        grid=(nblk,),
        in_specs=[
            pl.BlockSpec((_DB, Z), lambda i: (i, 0)),
            pl.BlockSpec((_DB, Z), lambda i: (i, 0)),
            pl.BlockSpec((_DB, 1), lambda i: (i, 0)),
            pl.BlockSpec((_DB, 1), lambda i: (i, 0)),
            pl.BlockSpec((Z, H), lambda i: (0, 0)),
            pl.BlockSpec((1, H), lambda i: (0, 0)),
            pl.BlockSpec((1, H), lambda i: (0, 0)),
            pl.BlockSpec((H, 1), lambda i: (0, 0)),
        ],
        out_specs=[
            pl.BlockSpec((_DB, 1), lambda i: (i, 0)),
            pl.BlockSpec((_DB, 1), lambda i: (i, 0)),
        ],
        out_shape=[jax.ShapeDtypeStruct((B, 1), jnp.float32),
                   jax.ShapeDtypeStruct((B, 1), jnp.float32)],
    )(zi, zj, tf, tcf, Wd1a, wlast, bd1, Wd2)


def kernel(edge_index, features, edges, T_f_batch, T_cf_batch,
           W1, b1, W2, b2, W3, b3, g1, be1, g2, be2, g3, be3,
           jk_w, Wd1, bd1, Wd2):
    pad = jnp.full((EPAD,), N, jnp.int32)
    srcr = jnp.concatenate([edge_index[0], pad]).reshape(NW, ENCH, EK)
    dstr = jnp.concatenate([edge_index[1], pad]).reshape(NW, ENCH, EK)

    h1 = _tc_matmul(features, W1)
    degp = _sc_degree(dstr).reshape(NC, N)
    degt = degp.T
    dinv, hs = _tc_prescale(h1, degt)

    b1r, g1r, be1r = b1.reshape(1, H), g1.reshape(1, H), be1.reshape(1, H)
    b2r, g2r, be2r = b2.reshape(1, H), g2.reshape(1, H), be2.reshape(1, H)
    b3r, g3r, be3r = b3.reshape(1, Z), g3.reshape(1, Z), be3.reshape(1, Z)

    acc = _sc_scatter(hs, srcr, dstr)
    o1, hs = _tc_mid(acc, hs, dinv, b1r, g1r, be1r, W2)
    acc = _sc_scatter(hs, srcr, dstr)
    o2, hs = _tc_mid(acc, hs, dinv, b2r, g2r, be2r, W3)
    acc = _sc_scatter(hs, srcr, dstr)
    z = _tc_final(acc, hs, dinv, b3r, g3r, be3r, o1, o2, jk_w.reshape(1, 3))

    eir = edges[:, 0].reshape(NW, BNCH, BK)
    ejr = edges[:, 1].reshape(NW, BNCH, BK)
    zi, zj = _sc_gather(z, eir, ejr)
    logits_f, logits_cf = _tc_decoder(
        zi, zj, T_f_batch.reshape(B, 1), T_cf_batch.reshape(B, 1),
        Wd1[:Z], Wd1[Z:Z + 1], bd1.reshape(1, H), Wd2)
    return (z, logits_f.reshape(B), logits_cf.reshape(B))


# speedup vs baseline: 1.1050x; 1.0768x over previous
import functools

import jax
import jax.numpy as jnp
from jax import lax
from jax.experimental import pallas as pl
from jax.experimental.pallas import tpu as pltpu, tpu_sc as plsc

N = 10000
E = 320000
D = 128
H = 128
Z = 128
B = 16384

NC = 2
NS = 16
NW = NC * NS

EK = 80
ENCH = 128
EPWP = ENCH * EK
EPAD = NW * EPWP - E
NP = N + 8

RPS = 624
ZR = 16

BPW = B // NW
BK = 128
BNCH = BPW // BK

NBUF = 3
CPP = 32
PASSES = ENCH // CPP

_mesh = plsc.VectorSubcoreMesh(core_axis_name="c", subcore_axis_name="s")


def _zero_vec():
    return jnp.zeros((16,), jnp.float32)


@functools.partial(
    pl.kernel,
    out_type=jax.ShapeDtypeStruct((NC * N,), jnp.float32),
    mesh=_mesh,
    scratch_types=[
        pltpu.VMEM_SHARED((NP,), jnp.float32),
        pltpu.VMEM((ENCH, EK), jnp.int32),
        pltpu.VMEM((EK,), jnp.float32),
        pltpu.VMEM((1024,), jnp.float32),
        pltpu.VMEM((1000,), jnp.float32),
        pltpu.SemaphoreType.DMA((4,)),
    ],
)
def _sc_degree(dstr_hbm, deg_hbm, acc_sh, dst_l, ones_v, zb_v, dg_v, ssem):
    NB = 4
    c = lax.axis_index("c")
    s = lax.axis_index("s")
    w = c * NS + s
    for k in range(64):
        zb_v[pl.ds(k * 16, 16)] = _zero_vec()
    for k in range(EK // 16):
        ones_v[pl.ds(k * 16, 16)] = jnp.ones((16,), jnp.float32)
    @pl.when(s < 10)
    def _():
        pltpu.sync_copy(zb_v.at[pl.ds(0, 1000)], acc_sh.at[pl.ds(s * 1000, 1000)])
    plsc.subcore_barrier()
    pltpu.sync_copy(dstr_hbm.at[w], dst_l)

    def body(j, carry):
        b = lax.rem(j, NB)

        @pl.when(j >= NB)
        def _():
            pltpu.make_async_copy(ones_v, acc_sh.at[dst_l.at[j - NB]],
                                  ssem.at[b]).wait()
        pltpu.async_copy(ones_v, acc_sh.at[dst_l.at[j]], ssem.at[b], add=True)
        return carry

    lax.fori_loop(0, ENCH, body, 0)
    for k in range(NB):
        j = ENCH - NB + k
        pltpu.make_async_copy(ones_v, acc_sh.at[dst_l.at[j]],
                              ssem.at[j % NB]).wait()
    plsc.subcore_barrier()

    @pl.when(s < 10)
    def _():
        pltpu.sync_copy(acc_sh.at[pl.ds(s * 1000, 1000)], dg_v)
        pltpu.sync_copy(dg_v, deg_hbm.at[pl.ds(c * N + s * 1000, 1000)])


@functools.partial(
    pl.kernel,
    out_type=jax.ShapeDtypeStruct((NC, N, D), jnp.float32),
    mesh=_mesh,
    scratch_types=[
        pltpu.VMEM_SHARED((NP, D), jnp.float32),
        pltpu.VMEM((CPP, EK), jnp.int32),
        pltpu.VMEM((CPP, EK), jnp.int32),
        pltpu.VMEM((NBUF, EK, D), jnp.float32),
        pltpu.VMEM((ZR, D), jnp.float32),
        pltpu.SemaphoreType.DMA((NBUF,)),
        pltpu.SemaphoreType.DMA((NBUF,)),
    ],
)
def _sc_scatter(hs_hbm, srcr_hbm, dstr_hbm, acc_hbm,
                acc_sh, src_l, dst_l, rows_v, zb_v, gsem, ssem):
    c = lax.axis_index("c")
    s = lax.axis_index("s")
    w = c * NS + s
    for i in range(ZR):
        for j in range(D // 16):
            zb_v[i, pl.ds(j * 16, 16)] = _zero_vec()

    base = s * RPS
    nch = jnp.where(s == NS - 1, 40, 39)

    def zbody(t, carry):
        pltpu.sync_copy(zb_v, acc_sh.at[pl.ds(base + t * ZR, ZR)])
        return carry
    lax.fori_loop(0, nch, zbody, 0)
    plsc.subcore_barrier()

    def pass_body(p, pcarry):
        pltpu.sync_copy(srcr_hbm.at[w, pl.ds(p * CPP, CPP)], src_l)
        pltpu.sync_copy(dstr_hbm.at[w, pl.ds(p * CPP, CPP)], dst_l)

        for b in range(NBUF - 1):
            pltpu.async_copy(hs_hbm.at[src_l.at[b]], rows_v.at[b], gsem.at[b])

        def body(j, carry):
            b = lax.rem(j, NBUF)
            c2 = j + NBUF - 1
            b2 = lax.rem(c2, NBUF)

            @pl.when(c2 < CPP)
            def _():
                @pl.when(j >= 1)
                def _():
                    pltpu.make_async_copy(rows_v.at[b2],
                                          acc_sh.at[dst_l.at[j - 1]],
                                          ssem.at[b2]).wait()
                pltpu.async_copy(hs_hbm.at[src_l.at[c2]], rows_v.at[b2],
                                 gsem.at[b2])
            pltpu.make_async_copy(hs_hbm.at[src_l.at[j]], rows_v.at[b],
                                  gsem.at[b]).wait()
            pltpu.async_copy(rows_v.at[b], acc_sh.at[dst_l.at[j]], ssem.at[b],
                             add=True)
            return carry

        lax.fori_loop(0, CPP, body, 0)
        for k in range(NBUF):
            j = CPP - NBUF + k
            pltpu.make_async_copy(rows_v.at[j % NBUF], acc_sh.at[dst_l.at[j]],
                                  ssem.at[j % NBUF]).wait()
        return pcarry

    lax.fori_loop(0, PASSES, pass_body, 0)
    plsc.subcore_barrier()

    def wbody(t, carry):
        pltpu.sync_copy(acc_sh.at[pl.ds(base + t * ZR, ZR)], zb_v)
        pltpu.sync_copy(zb_v, acc_hbm.at[c, pl.ds(base + t * ZR, ZR)])
        return carry
    lax.fori_loop(0, nch, wbody, 0)


_GNB = 4


@functools.partial(
    pl.kernel,
    out_type=[jax.ShapeDtypeStruct((B, Z), jnp.float32),
              jax.ShapeDtypeStruct((B, Z), jnp.float32)],
    mesh=_mesh,
    scratch_types=[
        pltpu.VMEM((2 * BNCH, BK), jnp.int32),
        pltpu.VMEM((_GNB, BK, Z), jnp.float32),
        pltpu.SemaphoreType.DMA((_GNB,)),
        pltpu.SemaphoreType.DMA((_GNB,)),
    ],
)
def _sc_gather(z_hbm, eir_hbm, ejr_hbm, zi_hbm, zj_hbm,
               idx_l, rows_v, gsem, wsem):
    c = lax.axis_index("c")
    s = lax.axis_index("s")
    w = c * NS + s
    pltpu.sync_copy(eir_hbm.at[w], idx_l.at[pl.ds(0, BNCH)])
    pltpu.sync_copy(ejr_hbm.at[w], idx_l.at[pl.ds(BNCH, BNCH)])

    NQ = 2 * BNCH

    def _out_slice(q):
        ref = zi_hbm if q < BNCH else zj_hbm
        return ref.at[pl.ds(w * BPW + (q % BNCH) * BK, BK)]

    for b in range(_GNB - 1):
        pltpu.async_copy(z_hbm.at[idx_l.at[b]], rows_v.at[b], gsem.at[b])
    for q in range(NQ):
        b = q % _GNB
        g = q + _GNB - 1
        if g < NQ:
            bg = g % _GNB
            if q >= 1:
                pltpu.make_async_copy(rows_v.at[bg], _out_slice(q - 1),
                                      wsem.at[bg]).wait()
            pltpu.async_copy(z_hbm.at[idx_l.at[g]], rows_v.at[bg],
                             gsem.at[bg])
        pltpu.make_async_copy(z_hbm.at[idx_l.at[q]], rows_v.at[b],
                              gsem.at[b]).wait()
        pltpu.async_copy(rows_v.at[b], _out_slice(q), wsem.at[b])
    for q in range(NQ - _GNB, NQ):
        pltpu.make_async_copy(rows_v.at[q % _GNB], _out_slice(q),
                              wsem.at[q % _GNB]).wait()


def _elu(x):
    return jnp.where(x > 0.0, x, jnp.exp(jnp.minimum(x, 0.0)) - 1.0)


def _tc_matmul_body(feat_ref, w1_ref, h_ref):
    h_ref[...] = jnp.dot(feat_ref[...], w1_ref[...],
                         preferred_element_type=jnp.float32)


def _tc_matmul(features, W1):
    return pl.pallas_call(
        _tc_matmul_body,
        out_shape=jax.ShapeDtypeStruct((N, D), jnp.float32),
    )(features, W1)


def _tc_prescale_body(h_ref, degt_ref, dinv_ref, hs1_ref):
    deg = degt_ref[:, 0:1] + degt_ref[:, 1:2] + 1.0
    dinv = lax.rsqrt(deg)
    dinv_ref[...] = dinv
    hs1_ref[0:N, :] = h_ref[...] * dinv
    hs1_ref[N:NP, :] = jnp.zeros((NP - N, D), jnp.float32)


def _tc_prescale(h, degt):
    return pl.pallas_call(
        _tc_prescale_body,
        out_shape=[jax.ShapeDtypeStruct((N, 1), jnp.float32),
                   jax.ShapeDtypeStruct((NP, D), jnp.float32)],
    )(h, degt)


def _combine(acc_ref, hs_ref, dinv_ref, b_ref, g_ref, be_ref):
    hs = hs_ref[...][0:N, :]
    pre = (acc_ref[0] + acc_ref[1] + hs) * dinv_ref[...] + b_ref[...]
    mean = jnp.mean(pre, axis=0, keepdims=True)
    var = jnp.mean((pre - mean) ** 2, axis=0, keepdims=True)
    o = (pre - mean) * lax.rsqrt(var + 1e-5) * g_ref[...] + be_ref[...]
    return _elu(o)


def _tc_mid_body(acc_ref, hs_ref, dinv_ref, b_ref, g_ref, be_ref, wn_ref,
                 o_ref, hsn_ref):
    o = _combine(acc_ref, hs_ref, dinv_ref, b_ref, g_ref, be_ref)
    o_ref[...] = o
    h = jnp.dot(o, wn_ref[...], preferred_element_type=jnp.float32)
    hsn_ref[0:N, :] = h * dinv_ref[...]
    hsn_ref[N:NP, :] = jnp.zeros((NP - N, D), jnp.float32)


def _tc_mid(acc, hs, dinv, b, g, be, Wn):
    return pl.pallas_call(
        _tc_mid_body,
        out_shape=[jax.ShapeDtypeStruct((N, D), jnp.float32),
                   jax.ShapeDtypeStruct((NP, D), jnp.float32)],
    )(acc, hs, dinv, b, g, be, Wn)


def _tc_final_body(acc_ref, hs_ref, dinv_ref, b_ref, g_ref, be_ref,
                   o1_ref, o2_ref, jk_ref, z_ref):
    o3 = _combine(acc_ref, hs_ref, dinv_ref, b_ref, g_ref, be_ref)
    jk = jk_ref[...]
    m = jnp.max(jk)
    e = jnp.exp(jk - m)
    wts = e / jnp.sum(e)
    z_ref[...] = (wts[0, 0] * o1_ref[...] + wts[0, 1] * o2_ref[...]
                  + wts[0, 2] * o3)


def _tc_final(acc, hs, dinv, b, g, be, o1, o2, jk_w):
    return pl.pallas_call(
        _tc_final_body,
        out_shape=jax.ShapeDtypeStruct((N, D), jnp.float32),
    )(acc, hs, dinv, b, g, be, o1, o2, jk_w)


_DB = 4096


def _tc_decoder_body(zi_ref, zj_ref, tf_ref, tcf_ref, wd1_ref, wlast_ref,
                     bd1_ref, wd2_ref, lf_ref, lcf_ref):
    zz = zi_ref[...] * zj_ref[...]
    gmat = jnp.dot(zz, wd1_ref[...], preferred_element_type=jnp.float32)
    gmat = gmat + bd1_ref[...]
    hf = _elu(gmat + tf_ref[...] * wlast_ref[...])
    hcf = _elu(gmat + tcf_ref[...] * wlast_ref[...])
    lf_ref[...] = jnp.dot(hf, wd2_ref[...], preferred_element_type=jnp.float32)
    lcf_ref[...] = jnp.dot(hcf, wd2_ref[...], preferred_element_type=jnp.float32)


def _tc_decoder(zi, zj, tf, tcf, Wd1a, wlast, bd1, Wd2):
    nblk = B // _DB
    return pl.pallas_call(
        _tc_decoder_body,
        grid=(nblk,),
        in_specs=[
            pl.BlockSpec((_DB, Z), lambda i: (i, 0)),
            pl.BlockSpec((_DB, Z), lambda i: (i, 0)),
            pl.BlockSpec((_DB, 1), lambda i: (i, 0)),
            pl.BlockSpec((_DB, 1), lambda i: (i, 0)),
            pl.BlockSpec((Z, H), lambda i: (0, 0)),
            pl.BlockSpec((1, H), lambda i: (0, 0)),
            pl.BlockSpec((1, H), lambda i: (0, 0)),
            pl.BlockSpec((H, 1), lambda i: (0, 0)),
        ],
        out_specs=[
            pl.BlockSpec((_DB, 1), lambda i: (i, 0)),
            pl.BlockSpec((_DB, 1), lambda i: (i, 0)),
        ],
        out_shape=[jax.ShapeDtypeStruct((B, 1), jnp.float32),
                   jax.ShapeDtypeStruct((B, 1), jnp.float32)],
    )(zi, zj, tf, tcf, Wd1a, wlast, bd1, Wd2)


def kernel(edge_index, features, edges, T_f_batch, T_cf_batch,
           W1, b1, W2, b2, W3, b3, g1, be1, g2, be2, g3, be3,
           jk_w, Wd1, bd1, Wd2):
    iota = jnp.arange(EPAD, dtype=jnp.int32)
    srcr = jnp.concatenate([edge_index[0], jnp.full((EPAD,), N, jnp.int32)]
                           ).reshape(NW, ENCH, EK)
    dstr = jnp.concatenate([edge_index[1], iota % N]).reshape(NW, ENCH, EK)
    dstd = jnp.concatenate([edge_index[1], N + (iota % 8)]
                           ).reshape(NW, ENCH, EK)

    h1 = _tc_matmul(features, W1)
    degp = _sc_degree(dstd).reshape(NC, N)
    degt = degp.T
    dinv, hs = _tc_prescale(h1, degt)

    b1r, g1r, be1r = b1.reshape(1, H), g1.reshape(1, H), be1.reshape(1, H)
    b2r, g2r, be2r = b2.reshape(1, H), g2.reshape(1, H), be2.reshape(1, H)
    b3r, g3r, be3r = b3.reshape(1, Z), g3.reshape(1, Z), be3.reshape(1, Z)

    acc = _sc_scatter(hs, srcr, dstr)
    o1, hs = _tc_mid(acc, hs, dinv, b1r, g1r, be1r, W2)
    acc = _sc_scatter(hs, srcr, dstr)
    o2, hs = _tc_mid(acc, hs, dinv, b2r, g2r, be2r, W3)
    acc = _sc_scatter(hs, srcr, dstr)
    z = _tc_final(acc, hs, dinv, b3r, g3r, be3r, o1, o2, jk_w.reshape(1, 3))

    eir = edges[:, 0].reshape(NW, BNCH, BK)
    ejr = edges[:, 1].reshape(NW, BNCH, BK)
    zi, zj = _sc_gather(z, eir, ejr)
    logits_f, logits_cf = _tc_decoder(
        zi, zj, T_f_batch.reshape(B, 1), T_cf_batch.reshape(B, 1),
        Wd1[:Z], Wd1[Z:Z + 1], bd1.reshape(1, H), Wd2)
    return (z, logits_f.reshape(B), logits_cf.reshape(B))


# speedup vs baseline: 3.5381x; 3.2020x over previous
import functools

import jax
import jax.numpy as jnp
from jax import lax
from jax.experimental import pallas as pl
from jax.experimental.pallas import tpu as pltpu, tpu_sc as plsc

N = 10000
E = 320000
D = 128
H = 128
Z = 128
B = 16384

NC = 2
NS = 16
NW = NC * NS

EK = 80
ENCH = 128
EPWP = ENCH * EK
EPAD = NW * EPWP - E
NP = N + 128

RPS = 624
ZR = 16

BPW = B // NW
BK = 128
BNCH = BPW // BK

NBUF = 3
CPP = 32
PASSES = ENCH // CPP

_mesh = plsc.VectorSubcoreMesh(core_axis_name="c", subcore_axis_name="s")


def _zero_vec():
    return jnp.zeros((16,), jnp.float32)


@functools.partial(
    pl.kernel,
    out_type=jax.ShapeDtypeStruct((NC * N,), jnp.float32),
    mesh=_mesh,
    scratch_types=[
        pltpu.VMEM_SHARED((NP,), jnp.float32),
        pltpu.VMEM((ENCH, EK), jnp.int32),
        pltpu.VMEM((EK,), jnp.float32),
        pltpu.VMEM((1024,), jnp.float32),
        pltpu.VMEM((1000,), jnp.float32),
        pltpu.SemaphoreType.DMA((4,)),
    ],
)
def _sc_degree(dstr_hbm, deg_hbm, acc_sh, dst_l, ones_v, zb_v, dg_v, ssem):
    NB = 4
    c = lax.axis_index("c")
    s = lax.axis_index("s")
    w = c * NS + s
    for k in range(64):
        zb_v[pl.ds(k * 16, 16)] = _zero_vec()
    for k in range(EK // 16):
        ones_v[pl.ds(k * 16, 16)] = jnp.ones((16,), jnp.float32)
    @pl.when(s < 10)
    def _():
        pltpu.sync_copy(zb_v.at[pl.ds(0, 1000)], acc_sh.at[pl.ds(s * 1000, 1000)])
    plsc.subcore_barrier()
    pltpu.sync_copy(dstr_hbm.at[w], dst_l)

    def body(j, carry):
        b = lax.rem(j, NB)

        @pl.when(j >= NB)
        def _():
            pltpu.make_async_copy(ones_v, acc_sh.at[dst_l.at[j - NB]],
                                  ssem.at[b]).wait()
        pltpu.async_copy(ones_v, acc_sh.at[dst_l.at[j]], ssem.at[b], add=True)
        return carry

    lax.fori_loop(0, ENCH, body, 0)
    for k in range(NB):
        j = ENCH - NB + k
        pltpu.make_async_copy(ones_v, acc_sh.at[dst_l.at[j]],
                              ssem.at[j % NB]).wait()
    plsc.subcore_barrier()

    @pl.when(s < 10)
    def _():
        pltpu.sync_copy(acc_sh.at[pl.ds(s * 1000, 1000)], dg_v)
        pltpu.sync_copy(dg_v, deg_hbm.at[pl.ds(c * N + s * 1000, 1000)])


@functools.partial(
    pl.kernel,
    out_type=jax.ShapeDtypeStruct((NC, N, D), jnp.float32),
    mesh=_mesh,
    scratch_types=[
        pltpu.VMEM_SHARED((NP, D), jnp.float32),
        pltpu.VMEM((CPP, EK), jnp.int32),
        pltpu.VMEM((CPP, EK), jnp.int32),
        pltpu.VMEM((NBUF, EK, D), jnp.float32),
        pltpu.VMEM((ZR, D), jnp.float32),
        pltpu.SemaphoreType.DMA((NBUF,)),
        pltpu.SemaphoreType.DMA((NBUF,)),
    ],
)
def _sc_scatter(hs_hbm, srcr_hbm, dstr_hbm, acc_hbm,
                acc_sh, src_l, dst_l, rows_v, zb_v, gsem, ssem):
    c = lax.axis_index("c")
    s = lax.axis_index("s")
    w = c * NS + s
    for i in range(ZR):
        for j in range(D // 16):
            zb_v[i, pl.ds(j * 16, 16)] = _zero_vec()

    base = s * RPS
    nch = jnp.where(s == NS - 1, 40, 39)

    def zbody(t, carry):
        pltpu.sync_copy(zb_v, acc_sh.at[pl.ds(base + t * ZR, ZR)])
        return carry
    lax.fori_loop(0, nch, zbody, 0)
    plsc.subcore_barrier()

    def pass_body(p, pcarry):
        pltpu.sync_copy(srcr_hbm.at[w, pl.ds(p * CPP, CPP)], src_l)
        pltpu.sync_copy(dstr_hbm.at[w, pl.ds(p * CPP, CPP)], dst_l)

        for b in range(NBUF - 1):
            pltpu.async_copy(hs_hbm.at[src_l.at[b]], rows_v.at[b], gsem.at[b])

        def body(j, carry):
            b = lax.rem(j, NBUF)
            c2 = j + NBUF - 1
            b2 = lax.rem(c2, NBUF)

            @pl.when(c2 < CPP)
            def _():
                @pl.when(j >= 1)
                def _():
                    pltpu.make_async_copy(rows_v.at[b2],
                                          acc_sh.at[dst_l.at[j - 1]],
                                          ssem.at[b2]).wait()
                pltpu.async_copy(hs_hbm.at[src_l.at[c2]], rows_v.at[b2],
                                 gsem.at[b2])
            pltpu.make_async_copy(hs_hbm.at[src_l.at[j]], rows_v.at[b],
                                  gsem.at[b]).wait()
            pltpu.async_copy(rows_v.at[b], acc_sh.at[dst_l.at[j]], ssem.at[b],
                             add=True)
            return carry

        lax.fori_loop(0, CPP, body, 0)
        for k in range(NBUF):
            j = CPP - NBUF + k
            pltpu.make_async_copy(rows_v.at[j % NBUF], acc_sh.at[dst_l.at[j]],
                                  ssem.at[j % NBUF]).wait()
        return pcarry

    lax.fori_loop(0, PASSES, pass_body, 0)
    plsc.subcore_barrier()

    def wbody(t, carry):
        pltpu.sync_copy(acc_sh.at[pl.ds(base + t * ZR, ZR)], zb_v)
        pltpu.sync_copy(zb_v, acc_hbm.at[c, pl.ds(base + t * ZR, ZR)])
        return carry
    lax.fori_loop(0, nch, wbody, 0)


_GNB = 4


@functools.partial(
    pl.kernel,
    out_type=[jax.ShapeDtypeStruct((B, Z), jnp.float32),
              jax.ShapeDtypeStruct((B, Z), jnp.float32)],
    mesh=_mesh,
    scratch_types=[
        pltpu.VMEM((2 * BNCH, BK), jnp.int32),
        pltpu.VMEM((_GNB, BK, Z), jnp.float32),
        pltpu.SemaphoreType.DMA((_GNB,)),
        pltpu.SemaphoreType.DMA((_GNB,)),
    ],
)
def _sc_gather(z_hbm, eir_hbm, ejr_hbm, zi_hbm, zj_hbm,
               idx_l, rows_v, gsem, wsem):
    c = lax.axis_index("c")
    s = lax.axis_index("s")
    w = c * NS + s
    pltpu.sync_copy(eir_hbm.at[w], idx_l.at[pl.ds(0, BNCH)])
    pltpu.sync_copy(ejr_hbm.at[w], idx_l.at[pl.ds(BNCH, BNCH)])

    NQ = 2 * BNCH

    def _out_slice(q):
        ref = zi_hbm if q < BNCH else zj_hbm
        return ref.at[pl.ds(w * BPW + (q % BNCH) * BK, BK)]

    for b in range(_GNB - 1):
        pltpu.async_copy(z_hbm.at[idx_l.at[b]], rows_v.at[b], gsem.at[b])
    for q in range(NQ):
        b = q % _GNB
        g = q + _GNB - 1
        if g < NQ:
            bg = g % _GNB
            if q >= 1:
                pltpu.make_async_copy(rows_v.at[bg], _out_slice(q - 1),
                                      wsem.at[bg]).wait()
            pltpu.async_copy(z_hbm.at[idx_l.at[g]], rows_v.at[bg],
                             gsem.at[bg])
        pltpu.make_async_copy(z_hbm.at[idx_l.at[q]], rows_v.at[b],
                              gsem.at[b]).wait()
        pltpu.async_copy(rows_v.at[b], _out_slice(q), wsem.at[b])
    for q in range(NQ - _GNB, NQ):
        pltpu.make_async_copy(rows_v.at[q % _GNB], _out_slice(q),
                              wsem.at[q % _GNB]).wait()


def _elu(x):
    return jnp.where(x > 0.0, x, jnp.exp(jnp.minimum(x, 0.0)) - 1.0)


def _tc_matmul_body(feat_ref, w1_ref, h_ref):
    h_ref[...] = jnp.dot(feat_ref[...], w1_ref[...],
                         preferred_element_type=jnp.float32)


def _tc_matmul(features, W1):
    return pl.pallas_call(
        _tc_matmul_body,
        out_shape=jax.ShapeDtypeStruct((N, D), jnp.float32),
    )(features, W1)


def _tc_prescale_body(h_ref, degt_ref, dinv_ref, hs1_ref):
    deg = degt_ref[:, 0:1] + degt_ref[:, 1:2] + 1.0
    dinv = lax.rsqrt(deg)
    dinv_ref[...] = dinv
    hs1_ref[0:N, :] = h_ref[...] * dinv
    hs1_ref[N:NP, :] = jnp.zeros((NP - N, D), jnp.float32)


def _tc_prescale(h, degt):
    return pl.pallas_call(
        _tc_prescale_body,
        out_shape=[jax.ShapeDtypeStruct((N, 1), jnp.float32),
                   jax.ShapeDtypeStruct((NP, D), jnp.float32)],
    )(h, degt)


def _combine(acc_ref, hs_ref, dinv_ref, b_ref, g_ref, be_ref):
    hs = hs_ref[...][0:N, :]
    pre = (acc_ref[0] + acc_ref[1] + hs) * dinv_ref[...] + b_ref[...]
    mean = jnp.mean(pre, axis=0, keepdims=True)
    var = jnp.mean((pre - mean) ** 2, axis=0, keepdims=True)
    o = (pre - mean) * lax.rsqrt(var + 1e-5) * g_ref[...] + be_ref[...]
    return _elu(o)


def _tc_mid_body(acc_ref, hs_ref, dinv_ref, b_ref, g_ref, be_ref, wn_ref,
                 o_ref, hsn_ref):
    o = _combine(acc_ref, hs_ref, dinv_ref, b_ref, g_ref, be_ref)
    o_ref[...] = o
    h = jnp.dot(o, wn_ref[...], preferred_element_type=jnp.float32)
    hsn_ref[0:N, :] = h * dinv_ref[...]
    hsn_ref[N:NP, :] = jnp.zeros((NP - N, D), jnp.float32)


def _tc_mid(acc, hs, dinv, b, g, be, Wn):
    return pl.pallas_call(
        _tc_mid_body,
        out_shape=[jax.ShapeDtypeStruct((N, D), jnp.float32),
                   jax.ShapeDtypeStruct((NP, D), jnp.float32)],
    )(acc, hs, dinv, b, g, be, Wn)


def _tc_final_body(acc_ref, hs_ref, dinv_ref, b_ref, g_ref, be_ref,
                   o1_ref, o2_ref, jk_ref, z_ref):
    o3 = _combine(acc_ref, hs_ref, dinv_ref, b_ref, g_ref, be_ref)
    jk = jk_ref[...]
    m = jnp.max(jk)
    e = jnp.exp(jk - m)
    wts = e / jnp.sum(e)
    z_ref[...] = (wts[0, 0] * o1_ref[...] + wts[0, 1] * o2_ref[...]
                  + wts[0, 2] * o3)


def _tc_final(acc, hs, dinv, b, g, be, o1, o2, jk_w):
    return pl.pallas_call(
        _tc_final_body,
        out_shape=jax.ShapeDtypeStruct((N, D), jnp.float32),
    )(acc, hs, dinv, b, g, be, o1, o2, jk_w)


_DB = 4096


def _tc_decoder_body(zi_ref, zj_ref, tf_ref, tcf_ref, wd1_ref, wlast_ref,
                     bd1_ref, wd2_ref, lf_ref, lcf_ref):
    zz = zi_ref[...] * zj_ref[...]
    gmat = jnp.dot(zz, wd1_ref[...], preferred_element_type=jnp.float32)
    gmat = gmat + bd1_ref[...]
    hf = _elu(gmat + tf_ref[...] * wlast_ref[...])
    hcf = _elu(gmat + tcf_ref[...] * wlast_ref[...])
    lf_ref[...] = jnp.dot(hf, wd2_ref[...], preferred_element_type=jnp.float32)
    lcf_ref[...] = jnp.dot(hcf, wd2_ref[...], preferred_element_type=jnp.float32)


def _tc_decoder(zi, zj, tf, tcf, Wd1a, wlast, bd1, Wd2):
    nblk = B // _DB
    return pl.pallas_call(
        _tc_decoder_body,
        grid=(nblk,),
        in_specs=[
            pl.BlockSpec((_DB, Z), lambda i: (i, 0)),
            pl.BlockSpec((_DB, Z), lambda i: (i, 0)),
            pl.BlockSpec((_DB, 1), lambda i: (i, 0)),
            pl.BlockSpec((_DB, 1), lambda i: (i, 0)),
            pl.BlockSpec((Z, H), lambda i: (0, 0)),
            pl.BlockSpec((1, H), lambda i: (0, 0)),
            pl.BlockSpec((1, H), lambda i: (0, 0)),
            pl.BlockSpec((H, 1), lambda i: (0, 0)),
        ],
        out_specs=[
            pl.BlockSpec((_DB, 1), lambda i: (i, 0)),
            pl.BlockSpec((_DB, 1), lambda i: (i, 0)),
        ],
        out_shape=[jax.ShapeDtypeStruct((B, 1), jnp.float32),
                   jax.ShapeDtypeStruct((B, 1), jnp.float32)],
    )(zi, zj, tf, tcf, Wd1a, wlast, bd1, Wd2)


def kernel(edge_index, features, edges, T_f_batch, T_cf_batch,
           W1, b1, W2, b2, W3, b3, g1, be1, g2, be2, g3, be3,
           jk_w, Wd1, bd1, Wd2):
    iota = jnp.arange(EPAD, dtype=jnp.int32)
    srcr = jnp.concatenate([edge_index[0], N + (iota % (NP - N))]
                           ).reshape(NW, ENCH, EK)
    dstr = jnp.concatenate([edge_index[1], iota % N]).reshape(NW, ENCH, EK)
    dstd = jnp.concatenate([edge_index[1], N + (iota % (NP - N))]
                           ).reshape(NW, ENCH, EK)

    h1 = _tc_matmul(features, W1)
    degp = _sc_degree(dstd).reshape(NC, N)
    degt = degp.T
    dinv, hs = _tc_prescale(h1, degt)

    b1r, g1r, be1r = b1.reshape(1, H), g1.reshape(1, H), be1.reshape(1, H)
    b2r, g2r, be2r = b2.reshape(1, H), g2.reshape(1, H), be2.reshape(1, H)
    b3r, g3r, be3r = b3.reshape(1, Z), g3.reshape(1, Z), be3.reshape(1, Z)

    acc = _sc_scatter(hs, srcr, dstr)
    o1, hs = _tc_mid(acc, hs, dinv, b1r, g1r, be1r, W2)
    acc = _sc_scatter(hs, srcr, dstr)
    o2, hs = _tc_mid(acc, hs, dinv, b2r, g2r, be2r, W3)
    acc = _sc_scatter(hs, srcr, dstr)
    z = _tc_final(acc, hs, dinv, b3r, g3r, be3r, o1, o2, jk_w.reshape(1, 3))

    eir = edges[:, 0].reshape(NW, BNCH, BK)
    ejr = edges[:, 1].reshape(NW, BNCH, BK)
    zi, zj = _sc_gather(z, eir, ejr)
    logits_f, logits_cf = _tc_decoder(
        zi, zj, T_f_batch.reshape(B, 1), T_cf_batch.reshape(B, 1),
        Wd1[:Z], Wd1[Z:Z + 1], bd1.reshape(1, H), Wd2)
    return (z, logits_f.reshape(B), logits_cf.reshape(B))
